# Initial kernel scaffold; baseline (speedup 1.0000x reference)
import jax
import jax.numpy as jnp
from jax.experimental import pallas as pl


def kernel(x, grid):
    raise NotImplementedError("write your pallas kernel here")


# speedup vs baseline: 1.5113x; 1.5113x over previous
import jax
import jax.numpy as jnp
from jax import lax
from jax.experimental import pallas as pl
from jax.experimental.pallas import tpu as pltpu
from jax.experimental.pallas import tpu_sc as plsc

N, C, H, W = 4, 3, 544, 960
HW = H * W
NPTS = N * HW
NC, NS = 2, 16
NW = NC * NS
PTS_W = NPTS // NW
WPS = NW // N
BLK = 768
NBLK = PTS_W // BLK
GCH = 128
NG = BLK // GCH


def _body(xp, gx_hbm, gy_hbm, out_hbm,
          gxv, gyv, i0v, i1v, w00v, w01v, w10v, w11v, val0, val1, outv, sem):
    cid = lax.axis_index("c")
    sid = lax.axis_index("s")
    wid = sid * NC + cid
    n = wid // WPS
    base = wid * PTS_W
    sample_off = (wid % WPS) * PTS_W
    tab_base = n * HW

    def blk(b, _):
        off = base + b * BLK
        pltpu.sync_copy(gx_hbm.at[pl.ds(off, BLK)], gxv)
        pltpu.sync_copy(gy_hbm.at[pl.ds(off, BLK)], gyv)

        def idxw(j, _):
            sl = pl.ds(j * 16, 16)
            gx = jnp.minimum(jnp.maximum(gxv[sl], -1.0), 1.0)
            gy = jnp.minimum(jnp.maximum(gyv[sl], -1.0), 1.0)
            ix = (gx + 1.0) * 0.5 * (W - 1)
            iy = (gy + 1.0) * 0.5 * (H - 1)
            x0 = ix.astype(jnp.int32)
            y0 = iy.astype(jnp.int32)
            wx1 = ix - x0.astype(jnp.float32)
            wy1 = iy - y0.astype(jnp.float32)
            wx0 = 1.0 - wx1
            wy0 = 1.0 - wy1
            row = y0 * W + x0 + tab_base
            i0v[sl] = row
            i1v[sl] = row + jnp.where(y0 < H - 1, W, 0)
            w00v[sl] = wy0 * wx0
            w01v[sl] = wy0 * wx1
            w10v[sl] = wy1 * wx0
            w11v[sl] = wy1 * wx1
            return _

        lax.fori_loop(0, BLK // 16, idxw, None)

        copies = []
        for k in range(NG):
            ks = pl.ds(k * GCH, GCH)
            copies.append(pltpu.async_copy(xp.at[i0v.at[ks]], val0.at[ks], sem))
            copies.append(pltpu.async_copy(xp.at[i1v.at[ks]], val1.at[ks], sem))
        for cp in copies:
            cp.wait()

        def comb(j, _):
            sl = pl.ds(j * 16, 16)
            rows = lax.iota(jnp.int32, 16) + j * 16
            w00 = w00v[sl]
            w01 = w01v[sl]
            w10 = w10v[sl]
            w11 = w11v[sl]
            for comp in range(C):
                c0 = jnp.full((16,), comp, jnp.int32)
                c1 = jnp.full((16,), comp + C, jnp.int32)
                v00 = plsc.load_gather(val0, [rows, c0])
                v01 = plsc.load_gather(val0, [rows, c1])
                v10 = plsc.load_gather(val1, [rows, c0])
                v11 = plsc.load_gather(val1, [rows, c1])
                outv[pl.ds(comp * BLK + j * 16, 16)] = (
                    (v00 * w00 + v01 * w01) + (v10 * w10 + v11 * w11))
            return _

        lax.fori_loop(0, BLK // 16, comb, None)

        for comp in range(C):
            o = (n * C + comp) * HW + sample_off + b * BLK
            pltpu.sync_copy(outv.at[pl.ds(comp * BLK, BLK)],
                            out_hbm.at[pl.ds(o, BLK)])
        return _

    lax.fori_loop(0, NBLK, blk, None)


_sc_call = pl.kernel(
    _body,
    out_type=jax.ShapeDtypeStruct((N * C * HW,), jnp.float32),
    mesh=plsc.VectorSubcoreMesh(
        core_axis_name="c", subcore_axis_name="s",
        num_cores=NC, num_subcores=NS),
    scratch_types=[
        pltpu.VMEM((BLK,), jnp.float32),
        pltpu.VMEM((BLK,), jnp.float32),
        pltpu.VMEM((BLK,), jnp.int32),
        pltpu.VMEM((BLK,), jnp.int32),
        pltpu.VMEM((BLK,), jnp.float32),
        pltpu.VMEM((BLK,), jnp.float32),
        pltpu.VMEM((BLK,), jnp.float32),
        pltpu.VMEM((BLK,), jnp.float32),
        pltpu.VMEM((BLK, 8), jnp.float32),
        pltpu.VMEM((BLK, 8), jnp.float32),
        pltpu.VMEM((C * BLK,), jnp.float32),
        pltpu.SemaphoreType.DMA,
    ],
    compiler_params=pltpu.CompilerParams(
        needs_layout_passes=False, use_tc_tiling_on_sc=False),
)


def kernel(x, grid):
    x_cl = jnp.transpose(x, (0, 2, 3, 1)).reshape(NPTS, C)
    nxt = jnp.concatenate([x_cl[1:], jnp.zeros((1, C), x_cl.dtype)], axis=0)
    pad = jnp.zeros((NPTS, 2), x_cl.dtype)
    xp = jnp.concatenate([x_cl, nxt, pad], axis=1)
    g = grid.reshape(NPTS, 2)
    out_flat = _sc_call(xp, g[:, 0], g[:, 1])
    return out_flat.reshape(N, C, H, W)
